# baseline (speedup 1.0000x reference)
import functools

import jax
import jax.numpy as jnp
from jax import lax
from jax.experimental import pallas as pl
from jax.experimental.pallas import tpu as pltpu
from jax.experimental.pallas import tpu_sc as plsc

N = 10000
E = 160000
D = 256
H = 512

NC = 2
NS = 16
K = 128

EP = 163840
NP = 10112
RPT = NP // NS
BN = 1000
NBLK = N // BN

_f32 = jnp.float32


def _make_edge_scatter(C):
  CPC = C // NC
  epw = EP // NS
  nb = epw // K

  nsb = nb // 4
  ngb = 2 * nsb
  KH = K // 2
  assert nsb % 2 == 0

  @functools.partial(
      pl.kernel,
      out_type=jax.ShapeDtypeStruct((C, NP, 128), _f32),
      mesh=plsc.VectorSubcoreMesh(core_axis_name="c", subcore_axis_name="s",
                                  num_cores=NC, num_subcores=NS),
      scratch_types=[
          pltpu.VMEM((ngb, KH), jnp.int32),
          pltpu.VMEM((nb, K), jnp.int32),
          pltpu.VMEM((K, 128), _f32),
          pltpu.VMEM((K, 128), _f32),
          pltpu.VMEM_SHARED((NP, 128), _f32),
          [[pltpu.SemaphoreType.DMA] * 2] * 2,
      ],
  )
  def k(g_hbm, src_hbm, dst_hbm, zeros_hbm, p_hbm, sidx, didx,
        rows0, rows1, acc, gsems):
    c = lax.axis_index("c")
    s = lax.axis_index("s")
    row0 = s * RPT
    rows = (rows0, rows1)
    pltpu.sync_copy(dst_hbm.at[s], didx)

    def fire(ci, i, b):
      for h in range(2):
        pltpu.async_copy(g_hbm.at[ci].at[sidx.at[2 * b + h]],
                         rows[i].at[pl.ds(h * KH, KH)], gsems[i][h])

    def drain(ci, i, b):
      for h in range(2):
        pltpu.make_async_copy(g_hbm.at[ci].at[sidx.at[2 * b + h]],
                              rows[i].at[pl.ds(h * KH, KH)],
                              gsems[i][h]).wait()

    for j in range(CPC):
      ci = c * CPC + j
      pltpu.sync_copy(zeros_hbm, acc.at[pl.ds(row0, RPT)])
      plsc.subcore_barrier()

      @pl.loop(0, nb // nsb)
      def _(p):
        pltpu.sync_copy(src_hbm.at[s].at[pl.ds(p * ngb, ngb)], sidx)
        b0 = p * nsb

        for i in range(2):
          fire(ci, i, i)

        @pl.loop(0, nsb // 2 - 1)
        def _(t):
          for i in range(2):
            b = 2 * t + i
            drain(ci, i, b)
            pltpu.sync_copy(rows[i], acc.at[didx.at[b0 + b]], add=True)
            fire(ci, i, b + 2)

        for i in range(2):
          b = nsb - 2 + i
          drain(ci, i, b)
          pltpu.sync_copy(rows[i], acc.at[didx.at[b0 + b]], add=True)

      plsc.subcore_barrier()
      pltpu.sync_copy(acc.at[pl.ds(row0, RPT)],
                      p_hbm.at[ci].at[pl.ds(row0, RPT)])

  return k


def _make_degree():
  nbt = EP // NS // K
  nbc = nbt // NC

  @functools.partial(
      pl.kernel,
      out_type=jax.ShapeDtypeStruct((NC, NP, 128), _f32),
      mesh=plsc.VectorSubcoreMesh(core_axis_name="c", subcore_axis_name="s",
                                  num_cores=NC, num_subcores=NS),
      scratch_types=[
          pltpu.VMEM((EP // NS // K, K), jnp.int32),
          pltpu.VMEM((K, 128), _f32),
          pltpu.VMEM_SHARED((NP, 128), _f32),
          pltpu.SemaphoreType.DMA,
      ],
  )
  def k(dst_hbm, ones_hbm, zeros_hbm, deg_hbm, didx, msg, acc, dsem):
    c = lax.axis_index("c")
    s = lax.axis_index("s")
    row0 = s * RPT
    pltpu.sync_copy(dst_hbm.at[s], didx)
    pltpu.sync_copy(ones_hbm, msg)
    pltpu.sync_copy(zeros_hbm, acc.at[pl.ds(row0, RPT)])
    plsc.subcore_barrier()

    @pl.loop(0, nbc)
    def _(t):
      pltpu.async_copy(msg, acc.at[didx.at[c * nbc + t]], dsem, add=True)

    @pl.loop(0, nbc)
    def _(t):
      pltpu.make_async_copy(msg, acc.at[didx.at[0]], dsem).wait()

    plsc.subcore_barrier()
    pltpu.sync_copy(acc.at[pl.ds(row0, RPT)],
                    deg_hbm.at[c].at[pl.ds(row0, RPT)])

  return k


_edge_scatter = functools.lru_cache(_make_edge_scatter)
_degree = functools.lru_cache(_make_degree)


def _dinv_body(deg_ref, o_ref):
  deg = 1.0 + deg_ref[0, :, 0:1] + deg_ref[1, :, 0:1]
  o_ref[...] = jnp.broadcast_to(lax.rsqrt(deg), (BN, 128))


def _dinv_call(degp):
  return pl.pallas_call(
      _dinv_body,
      grid=(NBLK,),
      in_specs=[pl.BlockSpec((NC, BN, 128), lambda i: (0, i, 0))],
      out_specs=pl.BlockSpec((BN, 128), lambda i: (i, 0)),
      out_shape=jax.ShapeDtypeStruct((N, 128), _f32),
  )(degp)


def _mm1_body(x_ref, w_ref, dinv_ref, o_ref):
  h = jnp.dot(x_ref[...], w_ref[...], preferred_element_type=_f32)
  o_ref[0] = dinv_ref[...] * h


def _mm1_call(x, w, dinv_b, c_chunks):
  kdim = x.shape[1]
  return pl.pallas_call(
      _mm1_body,
      grid=(c_chunks, NBLK),
      in_specs=[
          pl.BlockSpec((BN, kdim), lambda c, j: (j, 0)),
          pl.BlockSpec((kdim, 128), lambda c, j: (0, c)),
          pl.BlockSpec((BN, 128), lambda c, j: (j, 0)),
      ],
      out_specs=pl.BlockSpec((1, BN, 128), lambda c, j: (c, j, 0)),
      out_shape=jax.ShapeDtypeStruct((c_chunks, N, 128), _f32),
  )(x, w, dinv_b)


def _combine_relu_body(p_ref, g_ref, dinv_ref, b_ref, o_ref):
  v = dinv_ref[...] * (p_ref[0] + g_ref[0]) + b_ref[0]
  o_ref[0] = jnp.maximum(v, 0.0)


def _combine_relu_call(p, g, dinv_b, b_r, c_chunks):
  return pl.pallas_call(
      _combine_relu_body,
      grid=(c_chunks, NBLK),
      in_specs=[
          pl.BlockSpec((1, BN, 128), lambda c, j: (c, j, 0)),
          pl.BlockSpec((1, BN, 128), lambda c, j: (c, j, 0)),
          pl.BlockSpec((BN, 128), lambda c, j: (j, 0)),
          pl.BlockSpec((1, 1, 128), lambda c, j: (c, 0, 0)),
      ],
      out_specs=pl.BlockSpec((1, BN, 128), lambda c, j: (c, j, 0)),
      out_shape=jax.ShapeDtypeStruct((c_chunks, N, 128), _f32),
  )(p, g, dinv_b, b_r)


def _mm2_body(x_ref, w_ref, dinv_ref, o_ref, *, kc):
  kk = pl.program_id(2)

  @pl.when(kk == 0)
  def _():
    o_ref[0] = jnp.zeros_like(o_ref[0])

  o_ref[0] += jnp.dot(x_ref[0], w_ref[...], preferred_element_type=_f32)

  @pl.when(kk == kc - 1)
  def _():
    o_ref[0] = dinv_ref[...] * o_ref[0]


def _mm2_call(xc, w, dinv_b, c_chunks):
  kc = xc.shape[0]
  return pl.pallas_call(
      functools.partial(_mm2_body, kc=kc),
      grid=(c_chunks, NBLK, kc),
      in_specs=[
          pl.BlockSpec((1, BN, 128), lambda c, j, k: (k, j, 0)),
          pl.BlockSpec((128, 128), lambda c, j, k: (k, c)),
          pl.BlockSpec((BN, 128), lambda c, j, k: (j, 0)),
      ],
      out_specs=pl.BlockSpec((1, BN, 128), lambda c, j, k: (c, j, 0)),
      out_shape=jax.ShapeDtypeStruct((c_chunks, N, 128), _f32),
  )(xc, w, dinv_b)


def _final_body(p_ref, g_ref, dinv_ref, b_ref, o_ref):
  o_ref[...] = dinv_ref[...] * (p_ref[0] + g_ref[0]) + b_ref[0]


def _final_call(p, g, dinv_b, b_r, c_chunks):
  return pl.pallas_call(
      _final_body,
      grid=(c_chunks, NBLK),
      in_specs=[
          pl.BlockSpec((1, BN, 128), lambda c, j: (c, j, 0)),
          pl.BlockSpec((1, BN, 128), lambda c, j: (c, j, 0)),
          pl.BlockSpec((BN, 128), lambda c, j: (j, 0)),
          pl.BlockSpec((1, 1, 128), lambda c, j: (c, 0, 0)),
      ],
      out_specs=pl.BlockSpec((BN, 128), lambda c, j: (j, c)),
      out_shape=jax.ShapeDtypeStruct((N, c_chunks * 128), _f32),
  )(p, g, dinv_b, b_r)


def kernel(edge_index, emb, W1, b1, W2, b2):
  src = edge_index[0].astype(jnp.int32)
  dst = edge_index[1].astype(jnp.int32)
  pad = EP - E
  src_p = jnp.concatenate([src, jnp.zeros((pad,), jnp.int32)]).reshape(
      NS, EP // NS // (K // 2), K // 2)
  dst_p = jnp.concatenate([dst, jnp.full((pad,), N, jnp.int32)]).reshape(
      NS, EP // NS // K, K)

  zeros128 = jnp.zeros((RPT, 128), _f32)
  ones_msg = jnp.zeros((K, 128), _f32).at[:, 0].set(1.0)

  degp = _degree()(dst_p, ones_msg, zeros128)
  dinv_b = _dinv_call(degp)

  g1 = _mm1_call(emb, W1, dinv_b, H // 128)
  p1 = _edge_scatter(H // 128)(g1, src_p, dst_p, zeros128)
  x1 = _combine_relu_call(p1, g1, dinv_b, b1.reshape(H // 128, 1, 128),
                          H // 128)

  g2 = _mm2_call(x1, W2, dinv_b, D // 128)
  p2 = _edge_scatter(D // 128)(g2, src_p, dst_p, zeros128)
  out = _final_call(p2, g2, dinv_b, b2.reshape(D // 128, 1, 128), D // 128)
  return out


# speedup vs baseline: 1.0358x; 1.0358x over previous
import functools

import jax
import jax.numpy as jnp
from jax import lax
from jax.experimental import pallas as pl
from jax.experimental.pallas import tpu as pltpu
from jax.experimental.pallas import tpu_sc as plsc

N = 10000
E = 160000
D = 256
H = 512

NC = 2
NS = 16
K = 128

EP = 163840
NP = 10112
RPT = NP // NS
BN = 1000
NBLK = N // BN

_f32 = jnp.float32


def _make_edge_scatter(C):
  CPC = C // NC
  epw = EP // NS
  nb = epw // K

  nbh = nb // 2
  assert nbh % 2 == 0

  @functools.partial(
      pl.kernel,
      out_type=jax.ShapeDtypeStruct((C, NP, 128), _f32),
      mesh=plsc.VectorSubcoreMesh(core_axis_name="c", subcore_axis_name="s",
                                  num_cores=NC, num_subcores=NS),
      scratch_types=[
          pltpu.VMEM((nbh, K), jnp.int32),
          pltpu.VMEM((nbh, K), jnp.int32),
          pltpu.VMEM((K, 128), _f32),
          pltpu.VMEM((K, 128), _f32),
          pltpu.VMEM_SHARED((NP, 128), _f32),
          pltpu.SemaphoreType.DMA,
          pltpu.SemaphoreType.DMA,
      ],
  )
  def k(g_hbm, src_hbm, dst_hbm, zeros_hbm, p_hbm, sidx, didx,
        rows0, rows1, acc, gsem0, gsem1):
    c = lax.axis_index("c")
    s = lax.axis_index("s")
    row0 = s * RPT
    rows = (rows0, rows1)
    gsems = (gsem0, gsem1)
    for j in range(CPC):
      ci = c * CPC + j
      pltpu.sync_copy(zeros_hbm, acc.at[pl.ds(row0, RPT)])
      plsc.subcore_barrier()

      for p in range(2):
        pltpu.sync_copy(src_hbm.at[s].at[pl.ds(p * nbh, nbh)], sidx)
        pltpu.sync_copy(dst_hbm.at[s].at[pl.ds(p * nbh, nbh)], didx)

        for i in range(2):
          pltpu.async_copy(g_hbm.at[ci].at[sidx.at[i]], rows[i], gsems[i])

        @pl.loop(0, nbh // 2 - 1)
        def _(t):
          for i in range(2):
            b = 2 * t + i
            pltpu.make_async_copy(g_hbm.at[ci].at[sidx.at[b]],
                                  rows[i], gsems[i]).wait()
            pltpu.sync_copy(rows[i], acc.at[didx.at[b]], add=True)
            pltpu.async_copy(g_hbm.at[ci].at[sidx.at[b + 2]],
                             rows[i], gsems[i])

        for i in range(2):
          b = nbh - 2 + i
          pltpu.make_async_copy(g_hbm.at[ci].at[sidx.at[b]],
                                rows[i], gsems[i]).wait()
          pltpu.sync_copy(rows[i], acc.at[didx.at[b]], add=True)

      plsc.subcore_barrier()
      pltpu.sync_copy(acc.at[pl.ds(row0, RPT)],
                      p_hbm.at[ci].at[pl.ds(row0, RPT)])

  return k


def _make_degree():
  nbt = EP // NS // K
  nbc = nbt // NC

  @functools.partial(
      pl.kernel,
      out_type=jax.ShapeDtypeStruct((NC, NP, 128), _f32),
      mesh=plsc.VectorSubcoreMesh(core_axis_name="c", subcore_axis_name="s",
                                  num_cores=NC, num_subcores=NS),
      scratch_types=[
          pltpu.VMEM((EP // NS // K, K), jnp.int32),
          pltpu.VMEM((K, 128), _f32),
          pltpu.VMEM_SHARED((NP, 128), _f32),
          pltpu.SemaphoreType.DMA,
      ],
  )
  def k(dst_hbm, ones_hbm, zeros_hbm, deg_hbm, didx, msg, acc, dsem):
    c = lax.axis_index("c")
    s = lax.axis_index("s")
    row0 = s * RPT
    pltpu.sync_copy(dst_hbm.at[s], didx)
    pltpu.sync_copy(ones_hbm, msg)
    pltpu.sync_copy(zeros_hbm, acc.at[pl.ds(row0, RPT)])
    plsc.subcore_barrier()

    @pl.loop(0, nbc)
    def _(t):
      pltpu.async_copy(msg, acc.at[didx.at[c * nbc + t]], dsem, add=True)

    @pl.loop(0, nbc)
    def _(t):
      pltpu.make_async_copy(msg, acc.at[didx.at[0]], dsem).wait()

    plsc.subcore_barrier()
    pltpu.sync_copy(acc.at[pl.ds(row0, RPT)],
                    deg_hbm.at[c].at[pl.ds(row0, RPT)])

  return k


_edge_scatter = functools.lru_cache(_make_edge_scatter)
_degree = functools.lru_cache(_make_degree)


def _dinv_body(deg_ref, o_ref):
  deg = 1.0 + deg_ref[0, :, 0:1] + deg_ref[1, :, 0:1]
  o_ref[...] = jnp.broadcast_to(lax.rsqrt(deg), (BN, 128))


def _dinv_call(degp):
  return pl.pallas_call(
      _dinv_body,
      grid=(NBLK,),
      in_specs=[pl.BlockSpec((NC, BN, 128), lambda i: (0, i, 0))],
      out_specs=pl.BlockSpec((BN, 128), lambda i: (i, 0)),
      out_shape=jax.ShapeDtypeStruct((N, 128), _f32),
  )(degp)


def _mm1_body(x_ref, w_ref, dinv_ref, o_ref):
  h = jnp.dot(x_ref[...], w_ref[...], preferred_element_type=_f32)
  o_ref[0] = dinv_ref[...] * h


def _mm1_call(x, w, dinv_b, c_chunks):
  kdim = x.shape[1]
  return pl.pallas_call(
      _mm1_body,
      grid=(c_chunks, NBLK),
      in_specs=[
          pl.BlockSpec((BN, kdim), lambda c, j: (j, 0)),
          pl.BlockSpec((kdim, 128), lambda c, j: (0, c)),
          pl.BlockSpec((BN, 128), lambda c, j: (j, 0)),
      ],
      out_specs=pl.BlockSpec((1, BN, 128), lambda c, j: (c, j, 0)),
      out_shape=jax.ShapeDtypeStruct((c_chunks, N, 128), _f32),
  )(x, w, dinv_b)


def _mm2_body(p_ref, g_ref, w_ref, dinv_ref, b_ref, o_ref, *, kc):
  kk = pl.program_id(2)
  x = jnp.maximum(dinv_ref[...] * (p_ref[0] + g_ref[0]) + b_ref[0], 0.0)

  @pl.when(kk == 0)
  def _():
    o_ref[0] = jnp.zeros_like(o_ref[0])

  o_ref[0] += jnp.dot(x, w_ref[...], preferred_element_type=_f32)

  @pl.when(kk == kc - 1)
  def _():
    o_ref[0] = dinv_ref[...] * o_ref[0]


def _mm2_call(p, g, w, dinv_b, b_r, c_chunks):
  kc = g.shape[0]
  return pl.pallas_call(
      functools.partial(_mm2_body, kc=kc),
      grid=(c_chunks, NBLK, kc),
      in_specs=[
          pl.BlockSpec((1, BN, 128), lambda c, j, k: (k, j, 0)),
          pl.BlockSpec((1, BN, 128), lambda c, j, k: (k, j, 0)),
          pl.BlockSpec((128, 128), lambda c, j, k: (k, c)),
          pl.BlockSpec((BN, 128), lambda c, j, k: (j, 0)),
          pl.BlockSpec((1, 1, 128), lambda c, j, k: (k, 0, 0)),
      ],
      out_specs=pl.BlockSpec((1, BN, 128), lambda c, j, k: (c, j, 0)),
      out_shape=jax.ShapeDtypeStruct((c_chunks, N, 128), _f32),
  )(p, g, w, dinv_b, b_r)


def _final_body(p_ref, g_ref, dinv_ref, b_ref, o_ref):
  o_ref[...] = dinv_ref[...] * (p_ref[0] + g_ref[0]) + b_ref[0]


def _final_call(p, g, dinv_b, b_r, c_chunks):
  return pl.pallas_call(
      _final_body,
      grid=(c_chunks, NBLK),
      in_specs=[
          pl.BlockSpec((1, BN, 128), lambda c, j: (c, j, 0)),
          pl.BlockSpec((1, BN, 128), lambda c, j: (c, j, 0)),
          pl.BlockSpec((BN, 128), lambda c, j: (j, 0)),
          pl.BlockSpec((1, 1, 128), lambda c, j: (c, 0, 0)),
      ],
      out_specs=pl.BlockSpec((BN, 128), lambda c, j: (j, c)),
      out_shape=jax.ShapeDtypeStruct((N, c_chunks * 128), _f32),
  )(p, g, dinv_b, b_r)


def kernel(edge_index, emb, W1, b1, W2, b2):
  src = edge_index[0].astype(jnp.int32)
  dst = edge_index[1].astype(jnp.int32)
  pad = EP - E
  src_p = jnp.concatenate([src, jnp.zeros((pad,), jnp.int32)]).reshape(
      NS, EP // NS // K, K)
  dst_p = jnp.concatenate([dst, jnp.full((pad,), N, jnp.int32)]).reshape(
      NS, EP // NS // K, K)

  zeros128 = jnp.zeros((RPT, 128), _f32)
  ones_msg = jnp.zeros((K, 128), _f32).at[:, 0].set(1.0)

  degp = _degree()(dst_p, ones_msg, zeros128)
  dinv_b = _dinv_call(degp)

  g1 = _mm1_call(emb, W1, dinv_b, H // 128)
  p1 = _edge_scatter(H // 128)(g1, src_p, dst_p, zeros128)
  g2 = _mm2_call(p1, g1, W2, dinv_b, b1.reshape(H // 128, 1, 128), D // 128)
  p2 = _edge_scatter(D // 128)(g2, src_p, dst_p, zeros128)
  out = _final_call(p2, g2, dinv_b, b2.reshape(D // 128, 1, 128), D // 128)
  return out
